# baseline (speedup 1.0000x reference)
import functools

import jax
import jax.numpy as jnp
from jax import lax
from jax.experimental import pallas as pl
from jax.experimental.pallas import tpu as pltpu
from jax.experimental.pallas import tpu_sc as plsc

N = 10000
D = 128
E = 320000

NC = 2
NS = 16
CHUNK = 80
EDGES_PER_W = E // (NC * NS)
NCHUNKS = EDGES_PER_W // CHUNK
ACHUNK = 125
ANCH = EDGES_PER_W // ACHUNK
WB = 80
NROWCHUNKS = N // WB

_f32 = jnp.float32


def _zero_vmem_2d(ref, rows, cols):
    zv = jnp.zeros((16,), _f32)

    def body(i, _):
        r = i // (cols // 16)
        l = i - r * (cols // 16)
        ref[r, pl.ds(l * 16, 16)] = zv
        return 0

    lax.fori_loop(0, rows * (cols // 16), body, 0)


def _zero_vmem_1d(ref, n):
    zv = jnp.zeros((16,), _f32)

    def body(i, _):
        ref[pl.ds(i * 16, 16)] = zv
        return 0

    lax.fori_loop(0, n // 16, body, 0)


def _make_deg_kernel():
    mesh = plsc.VectorSubcoreMesh(core_axis_name="c", subcore_axis_name="s")

    @functools.partial(
        pl.kernel,
        mesh=mesh,
        out_type=jax.ShapeDtypeStruct((4, 1, N), _f32),
        scratch_types=[
            pltpu.VMEM((NCHUNKS, CHUNK), jnp.int32),
            pltpu.VMEM((NCHUNKS, CHUNK), jnp.int32),
            pltpu.VMEM((CHUNK,), _f32),
            pltpu.VMEM((N,), _f32),
            pltpu.VMEM_SHARED((N,), _f32),
            pltpu.VMEM_SHARED((N,), _f32),
        ]
        + [pltpu.SemaphoreType.DMA] * 8,
    )
    def deg_kernel(src_hbm, dst_hbm, out_hbm, sidx_v, didx_v, ones_v,
                   stage_v, dego_sh, degi_sh, *sems):
        sa = sems[:4]
        sb = sems[4:]
        c = lax.axis_index("c")
        s = lax.axis_index("s")
        w = c * NS + s

        ov = jnp.ones((16,), _f32)

        def fill_ones(i, _):
            ones_v[pl.ds(i * 16, 16)] = ov
            return 0

        lax.fori_loop(0, CHUNK // 16, fill_ones, 0)
        _zero_vmem_1d(stage_v, N)

        @pl.when(s == 0)
        def _():
            pltpu.sync_copy(stage_v, dego_sh)
            pltpu.sync_copy(stage_v, degi_sh)

        pltpu.sync_copy(src_hbm.at[w], sidx_v)
        pltpu.sync_copy(dst_hbm.at[w], didx_v)
        plsc.subcore_barrier()

        def issue(k, b):
            pltpu.async_copy(ones_v, dego_sh.at[sidx_v.at[k]], sa[b],
                             add=True)
            pltpu.async_copy(ones_v, degi_sh.at[didx_v.at[k]], sb[b],
                             add=True)

        def drain(b):
            pltpu.make_async_copy(ones_v, dego_sh.at[sidx_v.at[0]],
                                  sa[b]).wait()
            pltpu.make_async_copy(ones_v, degi_sh.at[didx_v.at[0]],
                                  sb[b]).wait()

        for b in range(4):
            issue(b, b)

        def body(g, _):
            for b in range(4):
                k = 4 * g + b
                drain(b)
                issue(k, b)
            return 0

        lax.fori_loop(1, NCHUNKS // 4, body, 0)
        drain(0)
        issue(NCHUNKS - 1, 0)
        for b in range(4):
            drain((b + 1) % 4)

        plsc.subcore_barrier()

        @pl.when(s == 0)
        def _():
            pltpu.sync_copy(dego_sh, stage_v)
            pltpu.sync_copy(stage_v, out_hbm.at[c * 2, 0])
            pltpu.sync_copy(degi_sh, stage_v)
            pltpu.sync_copy(stage_v, out_hbm.at[c * 2 + 1, 0])

    return deg_kernel


def _make_agg_kernel():
    mesh = plsc.VectorSubcoreMesh(core_axis_name="c", subcore_axis_name="s")

    @functools.partial(
        pl.kernel,
        mesh=mesh,
        out_type=jax.ShapeDtypeStruct((2, N, D), _f32),
        scratch_types=[
            pltpu.VMEM((2, 1, ACHUNK), jnp.int32),
            pltpu.VMEM((2, 1, ACHUNK), jnp.int32),
            pltpu.VMEM((2, ACHUNK, D), _f32),
            pltpu.VMEM_SHARED((N, D), _f32),
        ]
        + [pltpu.SemaphoreType.DMA] * 8,
    )
    def agg_kernel(h_hbm, src_hbm, dst_hbm, out_hbm, sidx_v, didx_v,
                   rows_v, acc_sh, *sems):
        gs = sems[:2]
        ss = sems[2:4]
        sis = sems[4:6]
        dis = sems[6:]
        c = lax.axis_index("c")
        s = lax.axis_index("s")
        w = c * NS + s

        stage = rows_v.at[0, pl.ds(0, WB)]
        zv = jnp.zeros((16,), _f32)

        def zero_body(i, _):
            r = i // (D // 16)
            l = i - r * (D // 16)
            rows_v[0, r, pl.ds(l * 16, 16)] = zv
            return 0

        lax.fori_loop(0, WB * (D // 16), zero_body, 0)
        nk = (NROWCHUNKS - s + NS - 1) // NS

        def zbody(k, _):
            pltpu.sync_copy(stage, acc_sh.at[pl.ds((s + k * NS) * WB, WB)])
            return 0

        lax.fori_loop(0, nk, zbody, 0)
        plsc.subcore_barrier()

        def si_issue(k, b):
            pltpu.async_copy(src_hbm.at[w, pl.ds(k, 1)], sidx_v.at[b],
                             sis[b])

        def si_wait(b):
            pltpu.make_async_copy(src_hbm.at[0, pl.ds(0, 1)], sidx_v.at[b],
                                  sis[b]).wait()

        def di_issue(k, b):
            pltpu.async_copy(dst_hbm.at[w, pl.ds(k, 1)], didx_v.at[b],
                             dis[b])

        def d_wait(b):
            pltpu.make_async_copy(dst_hbm.at[0, pl.ds(0, 1)], didx_v.at[b],
                                  dis[b]).wait()

        def g_issue(k, b):
            pltpu.async_copy(h_hbm.at[sidx_v.at[b, 0]], rows_v.at[b], gs[b])

        def g_wait(b):
            pltpu.make_async_copy(h_hbm.at[sidx_v.at[0, 0]], rows_v.at[b],
                                  gs[b]).wait()

        def s_issue(b):
            pltpu.async_copy(rows_v.at[b], acc_sh.at[didx_v.at[b, 0]], ss[b],
                             add=True)

        def s_wait(b):
            pltpu.make_async_copy(rows_v.at[b], acc_sh.at[didx_v.at[b, 0]],
                                  ss[b]).wait()

        def step(k, b, wait_prev=True, em_si=True, em_next=True):
            g_wait(b)
            d_wait(b)
            s_issue(b)
            if wait_prev:
                s_wait(1 - b)
            if em_si:
                si_issue(k + 2, b)
            if em_next:
                di_issue(k + 1, 1 - b)
                si_wait(1 - b)
                g_issue(k + 1, 1 - b)

        si_issue(0, 0)
        si_issue(1, 1)
        di_issue(0, 0)
        si_wait(0)
        g_issue(0, 0)
        step(0, 0, wait_prev=False)
        step(1, 1)

        def body(g, _):
            step(2 * g, 0)
            step(2 * g + 1, 1)
            return 0

        lax.fori_loop(1, ANCH // 2 - 1, body, 0)
        step(ANCH - 2, 0, em_si=False)
        step(ANCH - 1, 1, em_si=False, em_next=False)
        s_wait(1)

        plsc.subcore_barrier()

        def wb_body(k, _):
            r0 = (s + k * NS) * WB
            pltpu.sync_copy(acc_sh.at[pl.ds(r0, WB)], stage)
            pltpu.sync_copy(stage, out_hbm.at[c, pl.ds(r0, WB)])
            return 0

        lax.fori_loop(0, nk, wb_body, 0)

    return agg_kernel


_deg_kernel = _make_deg_kernel()
_agg_kernel = _make_agg_kernel()


BLK = 1000
GRID = N // BLK


def _tc_y_body(x_ref, w1_ref, y_ref):
    y_ref[...] = jnp.dot(x_ref[...], w1_ref[...],
                         preferred_element_type=_f32)


def _tc_y(x, W1):
    return pl.pallas_call(
        _tc_y_body,
        grid=(GRID,),
        in_specs=[
            pl.BlockSpec((BLK, D), lambda i: (i, 0)),
            pl.BlockSpec((D, D), lambda i: (0, 0)),
        ],
        out_specs=pl.BlockSpec((BLK, D), lambda i: (i, 0)),
        out_shape=jax.ShapeDtypeStruct((N, D), _f32),
    )(x, W1)


def _tc_a_body(y_ref, dp_ref, h0_ref, nrm_ref):
    dp = dp_ref[...]
    deg_out = dp[:, 0:1] + dp[:, 2:3]
    deg_in = dp[:, 1:2] + dp[:, 3:4]
    n_out = lax.rsqrt(jnp.maximum(deg_out, 1.0))
    n_in = lax.rsqrt(jnp.maximum(deg_in, 1.0))
    h0_ref[...] = y_ref[...] * n_out
    nrm_ref[...] = jnp.concatenate([n_out, n_in], axis=1)


def _tc_a(y, dpartT):
    return pl.pallas_call(
        _tc_a_body,
        grid=(GRID,),
        in_specs=[
            pl.BlockSpec((BLK, D), lambda i: (i, 0)),
            pl.BlockSpec((BLK, 4), lambda i: (i, 0)),
        ],
        out_specs=[
            pl.BlockSpec((BLK, D), lambda i: (i, 0)),
            pl.BlockSpec((BLK, 2), lambda i: (i, 0)),
        ],
        out_shape=[
            jax.ShapeDtypeStruct((N, D), _f32),
            jax.ShapeDtypeStruct((N, 2), _f32),
        ],
    )(y, dpartT)


def _tc_b_body(p0_ref, p1_ref, nrm_ref, w2_ref, b1_ref, out_ref):
    nrm = nrm_ref[...]
    m = (p0_ref[...] + p1_ref[...]) * nrm[:, 1:2] + b1_ref[...]
    h1 = jnp.maximum(m, 0.0)
    y = jnp.dot(h1, w2_ref[...], preferred_element_type=_f32)
    out_ref[...] = y * nrm[:, 0:1]


def _tc_b(p0, p1, nrm, W2, b1):
    return pl.pallas_call(
        _tc_b_body,
        grid=(GRID,),
        in_specs=[
            pl.BlockSpec((BLK, D), lambda i: (i, 0)),
            pl.BlockSpec((BLK, D), lambda i: (i, 0)),
            pl.BlockSpec((BLK, 2), lambda i: (i, 0)),
            pl.BlockSpec((D, D), lambda i: (0, 0)),
            pl.BlockSpec((1, D), lambda i: (0, 0)),
        ],
        out_specs=pl.BlockSpec((BLK, D), lambda i: (i, 0)),
        out_shape=jax.ShapeDtypeStruct((N, D), _f32),
    )(p0, p1, nrm, W2, b1)


def _tc_c_body(q0_ref, q1_ref, nrm_ref, b2_ref, wr_ref, br_ref, out_ref,
               acc_ref):
    i = pl.program_id(0)

    @pl.when(i == 0)
    def _():
        acc_ref[...] = jnp.zeros_like(acc_ref)

    nrm = nrm_ref[...]
    m = (q0_ref[...] + q1_ref[...]) * nrm[:, 1:2] + b2_ref[...]
    h2 = jnp.maximum(m, 0.0)
    acc_ref[...] += jnp.sum(h2, axis=0, keepdims=True)

    @pl.when(i == GRID - 1)
    def _():
        hg = acc_ref[...] * (1.0 / N)
        out_ref[...] = (
            jnp.dot(hg, wr_ref[...], preferred_element_type=_f32)
            + br_ref[...]
        )


def _tc_c(q0, q1, nrm, b2, Wr, br):
    return pl.pallas_call(
        _tc_c_body,
        grid=(GRID,),
        in_specs=[
            pl.BlockSpec((BLK, D), lambda i: (i, 0)),
            pl.BlockSpec((BLK, D), lambda i: (i, 0)),
            pl.BlockSpec((BLK, 2), lambda i: (i, 0)),
            pl.BlockSpec((1, D), lambda i: (0, 0)),
            pl.BlockSpec((D, 2), lambda i: (0, 0)),
            pl.BlockSpec((1, 2), lambda i: (0, 0)),
        ],
        out_specs=pl.BlockSpec((1, 2), lambda i: (0, 0)),
        out_shape=jax.ShapeDtypeStruct((1, 2), _f32),
        scratch_shapes=[pltpu.VMEM((1, D), _f32)],
    )(q0, q1, nrm, b2, Wr, br)


def kernel(x, edge_index, W1, b1, W2, b2, Wr, br):
    src = edge_index[0].astype(jnp.int32).reshape(NC * NS, NCHUNKS, CHUNK)
    dst = edge_index[1].astype(jnp.int32).reshape(NC * NS, NCHUNKS, CHUNK)
    srca = src.reshape(NC * NS, ANCH, ACHUNK)
    dsta = dst.reshape(NC * NS, ANCH, ACHUNK)

    y = _tc_y(x, W1)
    dpart = _deg_kernel(src, dst)
    dpartT = dpart.reshape(4, N).T
    h0, nrm = _tc_a(y, dpartT)

    p = _agg_kernel(h0, srca, dsta)
    h1s = _tc_b(p[0], p[1], nrm, W2, b1.reshape(1, D))

    q = _agg_kernel(h1s, srca, dsta)
    return _tc_c(q[0], q[1], nrm, b2.reshape(1, D), Wr, br.reshape(1, 2))


# speedup vs baseline: 1.2256x; 1.2256x over previous
import functools

import jax
import jax.numpy as jnp
from jax import lax
from jax.experimental import pallas as pl
from jax.experimental.pallas import tpu as pltpu
from jax.experimental.pallas import tpu_sc as plsc

N = 10000
D = 128
E = 320000

NC = 2
NS = 16
CHUNK = 80
EDGES_PER_W = E // (NC * NS)
NCHUNKS = EDGES_PER_W // CHUNK
WB = 80
NROWCHUNKS = N // WB

_f32 = jnp.float32


def _zero_vmem_2d(ref, rows, cols):
    zv = jnp.zeros((16,), _f32)

    def body(i, _):
        r = i // (cols // 16)
        l = i - r * (cols // 16)
        ref[r, pl.ds(l * 16, 16)] = zv
        return 0

    lax.fori_loop(0, rows * (cols // 16), body, 0)


def _zero_vmem_1d(ref, n):
    zv = jnp.zeros((16,), _f32)

    def body(i, _):
        ref[pl.ds(i * 16, 16)] = zv
        return 0

    lax.fori_loop(0, n // 16, body, 0)


def _make_deg_kernel():
    mesh = plsc.VectorSubcoreMesh(core_axis_name="c", subcore_axis_name="s")

    @functools.partial(
        pl.kernel,
        mesh=mesh,
        out_type=jax.ShapeDtypeStruct((4, 1, N), _f32),
        scratch_types=[
            pltpu.VMEM((NCHUNKS, CHUNK), jnp.int32),
            pltpu.VMEM((NCHUNKS, CHUNK), jnp.int32),
            pltpu.VMEM((CHUNK,), _f32),
            pltpu.VMEM((N,), _f32),
            pltpu.VMEM_SHARED((N,), _f32),
            pltpu.VMEM_SHARED((N,), _f32),
        ]
        + [pltpu.SemaphoreType.DMA] * 8,
    )
    def deg_kernel(src_hbm, dst_hbm, out_hbm, sidx_v, didx_v, ones_v,
                   stage_v, dego_sh, degi_sh, *sems):
        sa = sems[:4]
        sb = sems[4:]
        c = lax.axis_index("c")
        s = lax.axis_index("s")
        w = c * NS + s

        ov = jnp.ones((16,), _f32)

        def fill_ones(i, _):
            ones_v[pl.ds(i * 16, 16)] = ov
            return 0

        lax.fori_loop(0, CHUNK // 16, fill_ones, 0)
        _zero_vmem_1d(stage_v, N)

        @pl.when(s == 0)
        def _():
            pltpu.sync_copy(stage_v, dego_sh)
            pltpu.sync_copy(stage_v, degi_sh)

        pltpu.sync_copy(src_hbm.at[w], sidx_v)
        pltpu.sync_copy(dst_hbm.at[w], didx_v)
        plsc.subcore_barrier()

        def issue(k, b):
            pltpu.async_copy(ones_v, dego_sh.at[sidx_v.at[k]], sa[b],
                             add=True)
            pltpu.async_copy(ones_v, degi_sh.at[didx_v.at[k]], sb[b],
                             add=True)

        def drain(b):
            pltpu.make_async_copy(ones_v, dego_sh.at[sidx_v.at[0]],
                                  sa[b]).wait()
            pltpu.make_async_copy(ones_v, degi_sh.at[didx_v.at[0]],
                                  sb[b]).wait()

        for b in range(4):
            issue(b, b)

        def body(g, _):
            for b in range(4):
                k = 4 * g + b
                drain(b)
                issue(k, b)
            return 0

        lax.fori_loop(1, NCHUNKS // 4, body, 0)
        drain(0)
        issue(NCHUNKS - 1, 0)
        for b in range(4):
            drain((b + 1) % 4)

        plsc.subcore_barrier()

        @pl.when(s == 0)
        def _():
            pltpu.sync_copy(dego_sh, stage_v)
            pltpu.sync_copy(stage_v, out_hbm.at[c * 2, 0])
            pltpu.sync_copy(degi_sh, stage_v)
            pltpu.sync_copy(stage_v, out_hbm.at[c * 2 + 1, 0])

    return deg_kernel


def _make_agg_kernel():
    mesh = plsc.VectorSubcoreMesh(core_axis_name="c", subcore_axis_name="s")

    @functools.partial(
        pl.kernel,
        mesh=mesh,
        out_type=jax.ShapeDtypeStruct((2, N, D), _f32),
        scratch_types=[
            pltpu.VMEM((3, 1, CHUNK), jnp.int32),
            pltpu.VMEM((3, 1, CHUNK), jnp.int32),
            pltpu.VMEM((3, CHUNK, D), _f32),
            pltpu.VMEM_SHARED((N, D), _f32),
        ]
        + [pltpu.SemaphoreType.DMA] * 12,
    )
    def agg_kernel(h_hbm, src_hbm, dst_hbm, out_hbm, sidx_v, didx_v,
                   rows_v, acc_sh, *sems):
        gs = sems[:3]
        ss = sems[3:6]
        sis = sems[6:9]
        dis = sems[9:]
        c = lax.axis_index("c")
        s = lax.axis_index("s")
        w = c * NS + s

        stage = rows_v.at[0]
        zv = jnp.zeros((16,), _f32)

        def zero_body(i, _):
            r = i // (D // 16)
            l = i - r * (D // 16)
            rows_v[0, r, pl.ds(l * 16, 16)] = zv
            return 0

        lax.fori_loop(0, WB * (D // 16), zero_body, 0)
        nk = (NROWCHUNKS - s + NS - 1) // NS

        def zbody(k, _):
            pltpu.sync_copy(stage, acc_sh.at[pl.ds((s + k * NS) * WB, WB)])
            return 0

        lax.fori_loop(0, nk, zbody, 0)
        plsc.subcore_barrier()

        def si_issue(k, b):
            pltpu.async_copy(src_hbm.at[w, pl.ds(k, 1)], sidx_v.at[b],
                             sis[b])

        def si_wait(b):
            pltpu.make_async_copy(src_hbm.at[0, pl.ds(0, 1)], sidx_v.at[b],
                                  sis[b]).wait()

        def di_issue(k, b):
            pltpu.async_copy(dst_hbm.at[w, pl.ds(k, 1)], didx_v.at[b],
                             dis[b])

        def d_wait(b):
            pltpu.make_async_copy(dst_hbm.at[0, pl.ds(0, 1)], didx_v.at[b],
                                  dis[b]).wait()

        def g_issue(k, b):
            pltpu.async_copy(h_hbm.at[sidx_v.at[b, 0]], rows_v.at[b], gs[b])

        def g_wait(b):
            pltpu.make_async_copy(h_hbm.at[sidx_v.at[0, 0]], rows_v.at[b],
                                  gs[b]).wait()

        def s_issue(b):
            pltpu.async_copy(rows_v.at[b], acc_sh.at[didx_v.at[b, 0]], ss[b],
                             add=True)

        def s_wait(b):
            pltpu.make_async_copy(rows_v.at[b], acc_sh.at[didx_v.at[b, 0]],
                                  ss[b]).wait()

        def step(k, b, wait_prev=True, em_si=True, em_di=True, em_g=True):
            g_wait(b)
            d_wait(b)
            s_issue(b)
            if wait_prev:
                s_wait((b + 2) % 3)
            if em_si:
                si_issue(k + 3, b)
            if em_di:
                di_issue(k + 2, (b + 2) % 3)
            if em_g:
                si_wait((b + 2) % 3)
                g_issue(k + 2, (b + 2) % 3)

        si_issue(0, 0)
        si_issue(1, 1)
        si_issue(2, 2)
        di_issue(0, 0)
        di_issue(1, 1)
        si_wait(0)
        g_issue(0, 0)
        si_wait(1)
        g_issue(1, 1)
        step(0, 0, wait_prev=False)
        step(1, 1)
        step(2, 2)

        def body(g, _):
            step(3 * g, 0)
            step(3 * g + 1, 1)
            step(3 * g + 2, 2)
            return 0

        lax.fori_loop(1, 40, body, 0)
        step(120, 0)
        step(121, 1)
        step(122, 2, em_si=False)
        step(123, 0, em_si=False, em_di=False, em_g=False)
        step(124, 1, em_si=False, em_di=False, em_g=False)
        s_wait(1)

        plsc.subcore_barrier()

        def wb_body(k, _):
            r0 = (s + k * NS) * WB
            pltpu.sync_copy(acc_sh.at[pl.ds(r0, WB)], stage)
            pltpu.sync_copy(stage, out_hbm.at[c, pl.ds(r0, WB)])
            return 0

        lax.fori_loop(0, nk, wb_body, 0)

    return agg_kernel


_deg_kernel = _make_deg_kernel()
_agg_kernel = _make_agg_kernel()


BLK = 2000
GRID = N // BLK


def _tc_a_body(x_ref, w1_ref, dp_ref, h0_ref, nrm_ref):
    dp = dp_ref[...]
    deg_out = dp[:, 0:1] + dp[:, 2:3]
    deg_in = dp[:, 1:2] + dp[:, 3:4]
    n_out = lax.rsqrt(jnp.maximum(deg_out, 1.0))
    n_in = lax.rsqrt(jnp.maximum(deg_in, 1.0))
    y = jnp.dot(x_ref[...], w1_ref[...], preferred_element_type=_f32)
    h0_ref[...] = y * n_out
    nrm_ref[...] = jnp.concatenate([n_out, n_in], axis=1)


def _tc_a(x, W1, dpartT):
    return pl.pallas_call(
        _tc_a_body,
        grid=(GRID,),
        in_specs=[
            pl.BlockSpec((BLK, D), lambda i: (i, 0)),
            pl.BlockSpec((D, D), lambda i: (0, 0)),
            pl.BlockSpec((BLK, 4), lambda i: (i, 0)),
        ],
        out_specs=[
            pl.BlockSpec((BLK, D), lambda i: (i, 0)),
            pl.BlockSpec((BLK, 2), lambda i: (i, 0)),
        ],
        out_shape=[
            jax.ShapeDtypeStruct((N, D), _f32),
            jax.ShapeDtypeStruct((N, 2), _f32),
        ],
    )(x, W1, dpartT)


def _tc_b_body(p0_ref, p1_ref, nrm_ref, w2_ref, b1_ref, out_ref):
    nrm = nrm_ref[...]
    m = (p0_ref[...] + p1_ref[...]) * nrm[:, 1:2] + b1_ref[...]
    h1 = jnp.maximum(m, 0.0)
    y = jnp.dot(h1, w2_ref[...], preferred_element_type=_f32)
    out_ref[...] = y * nrm[:, 0:1]


def _tc_b(p0, p1, nrm, W2, b1):
    return pl.pallas_call(
        _tc_b_body,
        grid=(GRID,),
        in_specs=[
            pl.BlockSpec((BLK, D), lambda i: (i, 0)),
            pl.BlockSpec((BLK, D), lambda i: (i, 0)),
            pl.BlockSpec((BLK, 2), lambda i: (i, 0)),
            pl.BlockSpec((D, D), lambda i: (0, 0)),
            pl.BlockSpec((1, D), lambda i: (0, 0)),
        ],
        out_specs=pl.BlockSpec((BLK, D), lambda i: (i, 0)),
        out_shape=jax.ShapeDtypeStruct((N, D), _f32),
    )(p0, p1, nrm, W2, b1)


def _tc_c_body(q0_ref, q1_ref, nrm_ref, b2_ref, wr_ref, br_ref, out_ref,
               acc_ref):
    i = pl.program_id(0)

    @pl.when(i == 0)
    def _():
        acc_ref[...] = jnp.zeros_like(acc_ref)

    nrm = nrm_ref[...]
    m = (q0_ref[...] + q1_ref[...]) * nrm[:, 1:2] + b2_ref[...]
    h2 = jnp.maximum(m, 0.0)
    acc_ref[...] += jnp.sum(h2, axis=0, keepdims=True)

    @pl.when(i == GRID - 1)
    def _():
        hg = acc_ref[...] * (1.0 / N)
        out_ref[...] = (
            jnp.dot(hg, wr_ref[...], preferred_element_type=_f32)
            + br_ref[...]
        )


def _tc_c(q0, q1, nrm, b2, Wr, br):
    return pl.pallas_call(
        _tc_c_body,
        grid=(GRID,),
        in_specs=[
            pl.BlockSpec((BLK, D), lambda i: (i, 0)),
            pl.BlockSpec((BLK, D), lambda i: (i, 0)),
            pl.BlockSpec((BLK, 2), lambda i: (i, 0)),
            pl.BlockSpec((1, D), lambda i: (0, 0)),
            pl.BlockSpec((D, 2), lambda i: (0, 0)),
            pl.BlockSpec((1, 2), lambda i: (0, 0)),
        ],
        out_specs=pl.BlockSpec((1, 2), lambda i: (0, 0)),
        out_shape=jax.ShapeDtypeStruct((1, 2), _f32),
        scratch_shapes=[pltpu.VMEM((1, D), _f32)],
    )(q0, q1, nrm, b2, Wr, br)


def kernel(x, edge_index, W1, b1, W2, b2, Wr, br):
    src = edge_index[0].astype(jnp.int32).reshape(NC * NS, NCHUNKS, CHUNK)
    dst = edge_index[1].astype(jnp.int32).reshape(NC * NS, NCHUNKS, CHUNK)

    dpart = _deg_kernel(src, dst)
    dpartT = dpart.reshape(4, N).T
    h0, nrm = _tc_a(x, W1, dpartT)

    p = _agg_kernel(h0, src, dst)
    h1s = _tc_b(p[0], p[1], nrm, W2, b1.reshape(1, D))

    q = _agg_kernel(h1s, src, dst)
    return _tc_c(q[0], q[1], nrm, b2.reshape(1, D), Wr, br.reshape(1, 2))
